# baseline (speedup 1.0000x reference)
import jax
import jax.numpy as jnp
from jax.experimental import pallas as pl
from jax.experimental.pallas import tpu as pltpu


def _round_up(x, m):
    return ((x + m - 1) // m) * m


def _generator_kernel(adj_ref, fh_ref, fl_ref, wenc_ref, w2t_ref, bias_ref,
                      edges_ref, out_ref):
    a = adj_ref[...].astype(jnp.bfloat16)
    p = (jnp.dot(a, fh_ref[...], preferred_element_type=jnp.float32)
         + jnp.dot(a, fl_ref[...], preferred_element_type=jnp.float32))
    emb = jnp.tanh(jnp.dot(p, wenc_ref[...],
                           preferred_element_type=jnp.float32))
    s_all = jax.lax.dot_general(
        w2t_ref[...], emb, dimension_numbers=(((1,), (1,)), ((), ())),
        preferred_element_type=jnp.float32) + bias_ref[...]

    te = edges_ref.shape[-1]
    n = s_all.shape[1]
    nq = n // 128
    t0 = s_all[0:1, :].reshape(nq, 128)
    t1 = s_all[1:2, :].reshape(nq, 128)
    r_iota = jax.lax.broadcasted_iota(jnp.int32, (128, te), 0)
    q_iota = jax.lax.broadcasted_iota(jnp.int32, (nq, te), 0)

    def pick(tab, idx):
        r = jnp.bitwise_and(idx, 127)
        q = jnp.right_shift(idx, 7)
        ohr = (r_iota == r).astype(jnp.float32)
        u = jnp.dot(tab, ohr, preferred_element_type=jnp.float32)
        return jnp.sum(jnp.where(q_iota == q, u, 0.0), axis=0,
                       keepdims=True)

    out_ref[...] = (pick(t0, edges_ref[0:1, :])
                    + pick(t1, edges_ref[1:2, :]))


def kernel(feats, adj_stack, edge_stack, w_enc, weight_t, bias):
    n_views, n_nodes, _ = adj_stack.shape
    f = feats.shape[1]
    d = w_enc.shape[1]
    n_edges = edge_stack.shape[2]

    w2t = jnp.concatenate([weight_t[:d, :].T, weight_t[d:, :].T],
                          axis=0).astype(jnp.float32)
    bias2 = jnp.concatenate(
        [bias.reshape(1, 1).astype(jnp.float32),
         jnp.zeros((1, 1), jnp.float32)], axis=0)

    fh = feats.astype(jnp.bfloat16)
    fl = (feats - fh.astype(jnp.float32)).astype(jnp.bfloat16)

    e_pad = _round_up(n_edges, 128)
    edges_p = jnp.pad(edge_stack.astype(jnp.int32),
                      ((0, 0), (0, 0), (0, e_pad - n_edges)))

    out = pl.pallas_call(
        _generator_kernel,
        out_shape=jax.ShapeDtypeStruct((n_views, 1, e_pad), jnp.float32),
        grid=(n_views,),
        in_specs=[
            pl.BlockSpec((None, n_nodes, n_nodes), lambda vi: (vi, 0, 0)),
            pl.BlockSpec((n_nodes, f), lambda vi: (0, 0)),
            pl.BlockSpec((n_nodes, f), lambda vi: (0, 0)),
            pl.BlockSpec((f, d), lambda vi: (0, 0)),
            pl.BlockSpec((2, d), lambda vi: (0, 0)),
            pl.BlockSpec((2, 1), lambda vi: (0, 0)),
            pl.BlockSpec((None, 2, e_pad), lambda vi: (vi, 0, 0)),
        ],
        out_specs=pl.BlockSpec((None, 1, e_pad), lambda vi: (vi, 0, 0)),
        compiler_params=pltpu.CompilerParams(
            dimension_semantics=("parallel",),
            vmem_limit_bytes=100 * 1024 * 1024),
    )(adj_stack, fh, fl, w_enc.astype(jnp.float32), w2t, bias2, edges_p)

    logits = out[:, 0, :n_edges][..., None]
    return [logits[i] for i in range(n_views)]


# speedup vs baseline: 1.0949x; 1.0949x over previous
import jax
import jax.numpy as jnp
from jax.experimental import pallas as pl
from jax.experimental.pallas import tpu as pltpu


def _round_up(x, m):
    return ((x + m - 1) // m) * m


def _generator_kernel(adj_ref, fh_ref, fl_ref, wenc_ref, w2t_ref, bias_ref,
                      edges_ref, out_ref, s_ref):
    ni = pl.program_id(1)
    n_tiles = pl.num_programs(1)
    tm = adj_ref.shape[0]

    a = adj_ref[...].astype(jnp.bfloat16)
    p = (jnp.dot(a, fh_ref[...], preferred_element_type=jnp.float32)
         + jnp.dot(a, fl_ref[...], preferred_element_type=jnp.float32))
    emb = jnp.tanh(jnp.dot(p, wenc_ref[...],
                           preferred_element_type=jnp.float32))
    s_ref[:, pl.ds(ni * tm, tm)] = jax.lax.dot_general(
        w2t_ref[...], emb, dimension_numbers=(((1,), (1,)), ((), ())),
        preferred_element_type=jnp.float32) + bias_ref[...]

    @pl.when(ni == n_tiles - 1)
    def _():
        te = edges_ref.shape[-1]
        s_all = s_ref[...]
        n = s_all.shape[1]
        nq = n // 128
        t0 = s_all[0:1, :].reshape(nq, 128)
        t1 = s_all[1:2, :].reshape(nq, 128)
        r_iota = jax.lax.broadcasted_iota(jnp.int32, (128, te), 0)
        q_iota = jax.lax.broadcasted_iota(jnp.int32, (nq, te), 0)

        def pick(tab, idx):
            r = jnp.bitwise_and(idx, 127)
            q = jnp.right_shift(idx, 7)
            ohr = (r_iota == r).astype(jnp.float32)
            u = jnp.dot(tab, ohr,
                        preferred_element_type=jnp.float32)
            return jnp.sum(jnp.where(q_iota == q, u, 0.0), axis=0,
                           keepdims=True)

        out_ref[...] = (pick(t0, edges_ref[0:1, :])
                        + pick(t1, edges_ref[1:2, :]))


def kernel(feats, adj_stack, edge_stack, w_enc, weight_t, bias):
    n_views, n_nodes, _ = adj_stack.shape
    f = feats.shape[1]
    d = w_enc.shape[1]
    n_edges = edge_stack.shape[2]

    w2t = jnp.concatenate([weight_t[:d, :].T, weight_t[d:, :].T],
                          axis=0).astype(jnp.float32)
    bias2 = jnp.concatenate(
        [bias.reshape(1, 1).astype(jnp.float32),
         jnp.zeros((1, 1), jnp.float32)], axis=0)

    fh = feats.astype(jnp.bfloat16)
    fl = (feats - fh.astype(jnp.float32)).astype(jnp.bfloat16)

    e_pad = _round_up(n_edges, 128)
    edges_p = jnp.pad(edge_stack.astype(jnp.int32),
                      ((0, 0), (0, 0), (0, e_pad - n_edges)))

    tm = min(512, n_nodes)
    out = pl.pallas_call(
        _generator_kernel,
        out_shape=jax.ShapeDtypeStruct((n_views, 1, e_pad), jnp.float32),
        grid=(n_views, n_nodes // tm),
        in_specs=[
            pl.BlockSpec((None, tm, n_nodes), lambda vi, ni: (vi, ni, 0)),
            pl.BlockSpec((n_nodes, f), lambda vi, ni: (0, 0)),
            pl.BlockSpec((n_nodes, f), lambda vi, ni: (0, 0)),
            pl.BlockSpec((f, d), lambda vi, ni: (0, 0)),
            pl.BlockSpec((2, d), lambda vi, ni: (0, 0)),
            pl.BlockSpec((2, 1), lambda vi, ni: (0, 0)),
            pl.BlockSpec((None, 2, e_pad), lambda vi, ni: (vi, 0, 0)),
        ],
        out_specs=pl.BlockSpec((None, 1, e_pad), lambda vi, ni: (vi, 0, 0)),
        scratch_shapes=[
            pltpu.VMEM((2, n_nodes), jnp.float32),
        ],
        compiler_params=pltpu.CompilerParams(
            dimension_semantics=("parallel", "arbitrary"),
            vmem_limit_bytes=100 * 1024 * 1024),
    )(adj_stack, fh, fl, w_enc.astype(jnp.float32), w2t, bias2, edges_p)

    logits = out[:, 0, :n_edges][..., None]
    return [logits[i] for i in range(n_views)]


# speedup vs baseline: 1.1780x; 1.0759x over previous
import jax
import jax.numpy as jnp
from jax.experimental import pallas as pl
from jax.experimental.pallas import tpu as pltpu


def _round_up(x, m):
    return ((x + m - 1) // m) * m


def _generator_kernel(adj_ref, fh_ref, fl_ref, wenc_ref, w2t_ref, bias_ref,
                      edges_ref, out_ref, s_ref):
    ni = pl.program_id(1)
    n_tiles = pl.num_programs(1)
    tm = adj_ref.shape[0]

    a = adj_ref[...].astype(jnp.bfloat16)
    p = (jnp.dot(a, fh_ref[...], preferred_element_type=jnp.float32)
         + jnp.dot(a, fl_ref[...], preferred_element_type=jnp.float32))
    emb = jnp.tanh(jnp.dot(p, wenc_ref[...],
                           preferred_element_type=jnp.float32))
    s_ref[:, pl.ds(ni * tm, tm)] = jax.lax.dot_general(
        w2t_ref[...], emb, dimension_numbers=(((1,), (1,)), ((), ())),
        preferred_element_type=jnp.float32) + bias_ref[...]

    @pl.when(ni == n_tiles - 1)
    def _():
        te = edges_ref.shape[-1]
        s_all = s_ref[...]
        n = s_all.shape[1]
        nq = n // 128
        t0 = s_all[0:1, :].reshape(nq, 128)
        t1 = s_all[1:2, :].reshape(nq, 128)
        r_iota = jax.lax.broadcasted_iota(jnp.int32, (128, te), 0)
        q_iota = jax.lax.broadcasted_iota(jnp.int32, (nq, te), 0)

        def pick(tab, idx):
            r = jnp.bitwise_and(idx, 127)
            q = jnp.right_shift(idx, 7)
            ohr = (r_iota == r).astype(jnp.float32)
            u = jnp.dot(tab, ohr,
                        preferred_element_type=jnp.float32)
            return jnp.sum(jnp.where(q_iota == q, u, 0.0), axis=0,
                           keepdims=True)

        out_ref[...] = (pick(t0, edges_ref[0:1, :])
                        + pick(t1, edges_ref[1:2, :]))


def kernel(feats, adj_stack, edge_stack, w_enc, weight_t, bias):
    n_views, n_nodes, _ = adj_stack.shape
    f = feats.shape[1]
    d = w_enc.shape[1]
    n_edges = edge_stack.shape[2]

    w2t = jnp.concatenate([weight_t[:d, :].T, weight_t[d:, :].T],
                          axis=0).astype(jnp.float32)
    bias2 = jnp.concatenate(
        [bias.reshape(1, 1).astype(jnp.float32),
         jnp.zeros((1, 1), jnp.float32)], axis=0)

    fh = feats.astype(jnp.bfloat16)
    fl = (feats - fh.astype(jnp.float32)).astype(jnp.bfloat16)

    e_pad = _round_up(n_edges, 128)
    edges_p = jnp.pad(edge_stack.astype(jnp.int32),
                      ((0, 0), (0, 0), (0, e_pad - n_edges)))

    tm = min(1024, n_nodes)
    out = pl.pallas_call(
        _generator_kernel,
        out_shape=jax.ShapeDtypeStruct((n_views, 1, e_pad), jnp.float32),
        grid=(n_views, n_nodes // tm),
        in_specs=[
            pl.BlockSpec((None, tm, n_nodes), lambda vi, ni: (vi, ni, 0)),
            pl.BlockSpec((n_nodes, f), lambda vi, ni: (0, 0)),
            pl.BlockSpec((n_nodes, f), lambda vi, ni: (0, 0)),
            pl.BlockSpec((f, d), lambda vi, ni: (0, 0)),
            pl.BlockSpec((2, d), lambda vi, ni: (0, 0)),
            pl.BlockSpec((2, 1), lambda vi, ni: (0, 0)),
            pl.BlockSpec((None, 2, e_pad), lambda vi, ni: (vi, 0, 0)),
        ],
        out_specs=pl.BlockSpec((None, 1, e_pad), lambda vi, ni: (vi, 0, 0)),
        scratch_shapes=[
            pltpu.VMEM((2, n_nodes), jnp.float32),
        ],
        compiler_params=pltpu.CompilerParams(
            dimension_semantics=("parallel", "arbitrary"),
            vmem_limit_bytes=64 * 1024 * 1024),
    )(adj_stack, fh, fl, w_enc.astype(jnp.float32), w2t, bias2, edges_p)

    logits = out[:, 0, :n_edges][..., None]
    return [logits[i] for i in range(n_views)]


# speedup vs baseline: 1.3029x; 1.1061x over previous
import jax
import jax.numpy as jnp
from jax.experimental import pallas as pl
from jax.experimental.pallas import tpu as pltpu


def _generator_kernel(adj_ref, fh_ref, fl_ref, wenc_ref, wt_ref, bias_ref,
                      edges_ref, out_ref, s_ref):
    ni = pl.program_id(1)
    n_tiles = pl.num_programs(1)
    tm = adj_ref.shape[0]
    d = wenc_ref.shape[1]

    a = adj_ref[...].astype(jnp.bfloat16)
    p = (jnp.dot(a, fh_ref[...], preferred_element_type=jnp.float32)
         + jnp.dot(a, fl_ref[...], preferred_element_type=jnp.float32))
    emb = jnp.tanh(jnp.dot(p, wenc_ref[...],
                           preferred_element_type=jnp.float32))
    w2t = wt_ref[...].reshape(2, d)
    s_ref[:, pl.ds(ni * tm, tm)] = jax.lax.dot_general(
        w2t, emb, dimension_numbers=(((1,), (1,)), ((), ())),
        preferred_element_type=jnp.float32)

    @pl.when(ni == n_tiles - 1)
    def _():
        te = edges_ref.shape[-1]
        s_all = s_ref[...]
        n = s_all.shape[1]
        nq = n // 128
        t0 = s_all[0:1, :].reshape(nq, 128)
        t1 = s_all[1:2, :].reshape(nq, 128)
        r_iota = jax.lax.broadcasted_iota(jnp.int32, (128, te), 0)
        q_iota = jax.lax.broadcasted_iota(jnp.int32, (nq, te), 0)

        def pick(tab, idx):
            r = jnp.bitwise_and(idx, 127)
            q = jnp.right_shift(idx, 7)
            ohr = (r_iota == r).astype(jnp.float32)
            u = jnp.dot(tab, ohr,
                        preferred_element_type=jnp.float32)
            return jnp.sum(jnp.where(q_iota == q, u, 0.0), axis=0,
                           keepdims=True)

        out_ref[...] = (pick(t0, edges_ref[0:1, :])
                        + pick(t1, edges_ref[1:2, :])
                        + bias_ref[...])


def kernel(feats, adj_stack, edge_stack, w_enc, weight_t, bias):
    n_views, n_nodes, _ = adj_stack.shape
    f = feats.shape[1]
    d = w_enc.shape[1]
    n_edges = edge_stack.shape[2]

    fh = feats.astype(jnp.bfloat16)
    fl = (feats - fh.astype(jnp.float32)).astype(jnp.bfloat16)

    tm = min(1024, n_nodes)
    out = pl.pallas_call(
        _generator_kernel,
        out_shape=jax.ShapeDtypeStruct((n_views, 1, n_edges), jnp.float32),
        grid=(n_views, n_nodes // tm),
        in_specs=[
            pl.BlockSpec((None, tm, n_nodes), lambda vi, ni: (vi, ni, 0)),
            pl.BlockSpec((n_nodes, f), lambda vi, ni: (0, 0)),
            pl.BlockSpec((n_nodes, f), lambda vi, ni: (0, 0)),
            pl.BlockSpec((f, d), lambda vi, ni: (0, 0)),
            pl.BlockSpec((2 * d, 1), lambda vi, ni: (0, 0)),
            pl.BlockSpec((1, 1), lambda vi, ni: (0, 0)),
            pl.BlockSpec((None, 2, n_edges), lambda vi, ni: (vi, 0, 0)),
        ],
        out_specs=pl.BlockSpec((None, 1, n_edges), lambda vi, ni: (vi, 0, 0)),
        scratch_shapes=[
            pltpu.VMEM((2, n_nodes), jnp.float32),
        ],
        compiler_params=pltpu.CompilerParams(
            dimension_semantics=("parallel", "arbitrary"),
            vmem_limit_bytes=64 * 1024 * 1024),
    )(adj_stack, fh, fl, w_enc, weight_t, bias.astype(jnp.float32),
      edge_stack.astype(jnp.int32))

    logits = out[:, 0, :][..., None]
    return [logits[i] for i in range(n_views)]


# speedup vs baseline: 1.5675x; 1.2031x over previous
import jax
import jax.numpy as jnp
from jax.experimental import pallas as pl
from jax.experimental.pallas import tpu as pltpu


def _generator_kernel(adj_ref, feats_ref, wenc_ref, wt_ref, bias_ref,
                      edges_ref, out_ref, s_ref):
    ni = pl.program_id(1)
    n_tiles = pl.num_programs(1)
    tm = adj_ref.shape[0]
    d = wenc_ref.shape[1]

    p = jnp.dot(adj_ref[...], feats_ref[...],
                preferred_element_type=jnp.float32)
    emb = jnp.tanh(jnp.dot(p, wenc_ref[...],
                           preferred_element_type=jnp.float32))
    s_ref[:, pl.ds(ni * tm, tm)] = jax.lax.dot_general(
        wt_ref[...], emb, dimension_numbers=(((1,), (1,)), ((), ())),
        preferred_element_type=jnp.float32) + bias_ref[...]

    @pl.when(ni == n_tiles - 1)
    def _():
        te = edges_ref.shape[-1]
        s_all = s_ref[...]
        n = s_all.shape[1]
        nq = n // 128
        t0 = s_all[0:1, :].reshape(nq, 128)
        t1 = s_all[1:2, :].reshape(nq, 128)
        r_iota = jax.lax.broadcasted_iota(jnp.int32, (128, te), 0)
        q_iota = jax.lax.broadcasted_iota(jnp.int32, (nq, te), 0)

        def pick(tab, idx):
            r = jnp.bitwise_and(idx, 127)
            q = jnp.right_shift(idx, 7)
            ohr = (r_iota == r).astype(jnp.float32)
            u = jnp.dot(tab, ohr,
                        preferred_element_type=jnp.float32)
            return jnp.sum(jnp.where(q_iota == q, u, 0.0), axis=0,
                           keepdims=True)

        out_ref[...] = (pick(t0, edges_ref[0:1, :])
                        + pick(t1, edges_ref[1:2, :]))


def kernel(feats, adj_stack, edge_stack, w_enc, weight_t, bias):
    n_views, n_nodes, _ = adj_stack.shape
    f = feats.shape[1]
    d = w_enc.shape[1]
    n_edges = edge_stack.shape[2]

    w2t = jnp.concatenate([weight_t[:d, :].T, weight_t[d:, :].T],
                          axis=0).astype(jnp.float32)
    bias2 = jnp.concatenate(
        [bias.reshape(1, 1).astype(jnp.float32),
         jnp.zeros((1, 1), jnp.float32)], axis=0)

    tm = min(1024, n_nodes)
    out = pl.pallas_call(
        _generator_kernel,
        out_shape=jax.ShapeDtypeStruct((n_views, 1, n_edges), jnp.float32),
        grid=(n_views, n_nodes // tm),
        in_specs=[
            pl.BlockSpec((None, tm, n_nodes), lambda vi, ni: (vi, ni, 0)),
            pl.BlockSpec((n_nodes, f), lambda vi, ni: (0, 0)),
            pl.BlockSpec((f, d), lambda vi, ni: (0, 0)),
            pl.BlockSpec((2, d), lambda vi, ni: (0, 0)),
            pl.BlockSpec((2, 1), lambda vi, ni: (0, 0)),
            pl.BlockSpec((None, 2, n_edges), lambda vi, ni: (vi, 0, 0)),
        ],
        out_specs=pl.BlockSpec((None, 1, n_edges), lambda vi, ni: (vi, 0, 0)),
        scratch_shapes=[
            pltpu.VMEM((2, n_nodes), jnp.float32),
        ],
        compiler_params=pltpu.CompilerParams(
            dimension_semantics=("parallel", "arbitrary"),
            vmem_limit_bytes=64 * 1024 * 1024),
    )(adj_stack, feats, w_enc, w2t, bias2,
      edge_stack.astype(jnp.int32))

    logits = out[:, 0, :][..., None]
    return [logits[i] for i in range(n_views)]
